# Initial kernel scaffold; baseline (speedup 1.0000x reference)
import jax
import jax.numpy as jnp
from jax.experimental import pallas as pl


def kernel(x_src, x_dst, edge_attr, edge_index, q, kW0, kb0, kW1, kb1, vW0, vb0, vW1, vb1, oW0, ob0, oW1, ob1):
    raise NotImplementedError("write your pallas kernel here")


# speedup vs baseline: 3.4326x; 3.4326x over previous
import functools

import jax
import jax.numpy as jnp
import numpy as np
from jax import lax
from jax.experimental import pallas as pl
from jax.experimental.pallas import tpu as pltpu
from jax.experimental.pallas import tpu_sc as plsc

N_SRC = 10000
N_DST = 10000
E = 320000
D = 128
D_EDGE = 16
HEADS = 8
D_HEAD = 16
TW = 2 * D

NC, NS = 2, 16
NW = NC * NS
EPW = E // NW
EPS = E // NS

C1 = 80
C2 = 80
ZC = 80
NZCH = N_DST // ZC


def _tables_body(xs_ref, xd_ref, ws_ref, wd_ref, bd_ref, ts_ref, td_ref):
    ts_ref[...] = jnp.dot(xs_ref[...], ws_ref[...],
                          preferred_element_type=jnp.float32)
    td_ref[...] = jnp.dot(xd_ref[...], wd_ref[...],
                          preferred_element_type=jnp.float32) + bd_ref[...]


def _tc_tables(x_src, x_dst, w_src, w_dst, b_dst):
    blk = 1000
    grid = (N_SRC // blk,)
    return pl.pallas_call(
        _tables_body,
        grid=grid,
        in_specs=[
            pl.BlockSpec((blk, D), lambda i: (i, 0)),
            pl.BlockSpec((blk, D), lambda i: (i, 0)),
            pl.BlockSpec((D, TW), lambda i: (0, 0)),
            pl.BlockSpec((D, TW), lambda i: (0, 0)),
            pl.BlockSpec((1, TW), lambda i: (0, 0)),
        ],
        out_specs=[
            pl.BlockSpec((blk, TW), lambda i: (i, 0)),
            pl.BlockSpec((blk, TW), lambda i: (i, 0)),
        ],
        out_shape=[
            jax.ShapeDtypeStruct((N_SRC, TW), jnp.float32),
            jax.ShapeDtypeStruct((N_DST, TW), jnp.float32),
        ],
    )(x_src, x_dst, w_src, w_dst, b_dst)


def _sc_gather_body(ts_hbm, td_hbm, src_hbm, dst_hbm, gs_hbm, gd_hbm,
                    idx_v, rows_v, sem):
    wid = lax.axis_index("s") * NC + lax.axis_index("c")

    def body(i, _):
        base = wid * EPW + i * C1
        pltpu.sync_copy(src_hbm.at[pl.ds(base, C1)], idx_v)
        pltpu.async_copy(ts_hbm.at[idx_v], rows_v, sem).wait()
        pltpu.sync_copy(rows_v, gs_hbm.at[pl.ds(base, C1)])
        pltpu.sync_copy(dst_hbm.at[pl.ds(base, C1)], idx_v)
        pltpu.async_copy(td_hbm.at[idx_v], rows_v, sem).wait()
        pltpu.sync_copy(rows_v, gd_hbm.at[pl.ds(base, C1)])
        return 0

    lax.fori_loop(0, EPW // C1, body, 0)


def _sc_gather(ts, td, src, dst):
    mesh = plsc.VectorSubcoreMesh(core_axis_name="c", subcore_axis_name="s")
    f = functools.partial(
        pl.kernel,
        mesh=mesh,
        out_type=[
            jax.ShapeDtypeStruct((E, TW), jnp.float32),
            jax.ShapeDtypeStruct((E, TW), jnp.float32),
        ],
        scratch_types=[
            pltpu.VMEM((C1,), jnp.int32),
            pltpu.VMEM((C1, TW), jnp.float32),
            pltpu.SemaphoreType.DMA,
        ],
    )(_sc_gather_body)
    return f(ts, td, src, dst)


def _edge_body(gs_ref, gd_ref, ea_ref, wke_ref, wve_ref, qk_ref, ck_ref,
               wv1_ref, bv1_ref, rep_ref, mn_ref, md_ref):
    ea = ea_ref[...]
    h0k = jnp.maximum(
        gs_ref[:, :D] + gd_ref[:, :D]
        + jnp.dot(ea, wke_ref[...], preferred_element_type=jnp.float32), 0.0)
    coef = jnp.dot(h0k, qk_ref[...],
                   preferred_element_type=jnp.float32) + ck_ref[...]
    ex = jnp.exp(coef)
    h0v = jnp.maximum(
        gs_ref[:, D:] + gd_ref[:, D:]
        + jnp.dot(ea, wve_ref[...], preferred_element_type=jnp.float32), 0.0)
    v1 = jnp.dot(h0v, wv1_ref[...],
                 preferred_element_type=jnp.float32) + bv1_ref[...] + h0v
    exw = jnp.dot(ex, rep_ref[...],
                  preferred_element_type=jnp.float32)
    mn_ref[...] = exw * v1
    md_ref[...] = jnp.concatenate(
        [ex, jnp.zeros((ex.shape[0], D - HEADS), jnp.float32)], axis=1)


def _tc_edges(gs, gd, ea, wke, wve, qk, ck, wv1, bv1, rep):
    blk = 1000
    grid = (E // blk,)
    return pl.pallas_call(
        _edge_body,
        grid=grid,
        in_specs=[
            pl.BlockSpec((blk, TW), lambda i: (i, 0)),
            pl.BlockSpec((blk, TW), lambda i: (i, 0)),
            pl.BlockSpec((blk, D_EDGE), lambda i: (i, 0)),
            pl.BlockSpec((D_EDGE, D), lambda i: (0, 0)),
            pl.BlockSpec((D_EDGE, D), lambda i: (0, 0)),
            pl.BlockSpec((D, HEADS), lambda i: (0, 0)),
            pl.BlockSpec((1, HEADS), lambda i: (0, 0)),
            pl.BlockSpec((D, D), lambda i: (0, 0)),
            pl.BlockSpec((1, D), lambda i: (0, 0)),
            pl.BlockSpec((HEADS, D), lambda i: (0, 0)),
        ],
        out_specs=[
            pl.BlockSpec((blk, D), lambda i: (i, 0)),
            pl.BlockSpec((blk, D), lambda i: (i, 0)),
        ],
        out_shape=[
            jax.ShapeDtypeStruct((E, D), jnp.float32),
            jax.ShapeDtypeStruct((E, D), jnp.float32),
        ],
    )(gs, gd, ea, wke, wve, qk, ck, wv1, bv1, rep)


def _sc_scatter_body(mn_hbm, md_hbm, dst_hbm, z_hbm, out_hbm,
                     idx_v, rows_v, acc):
    c = lax.axis_index("c")
    s = lax.axis_index("s")

    pltpu.sync_copy(z_hbm, rows_v)

    def zinit(j, _):
        @pl.when(j % NS == s)
        def _():
            pltpu.sync_copy(rows_v, acc.at[pl.ds(j * ZC, ZC)])
        return 0

    lax.fori_loop(0, NZCH, zinit, 0)
    plsc.subcore_barrier()

    def body(i, _):
        base = s * EPS + i * C2
        pltpu.sync_copy(dst_hbm.at[pl.ds(base, C2)], idx_v)

        @pl.when(c == 0)
        def _():
            pltpu.sync_copy(mn_hbm.at[pl.ds(base, C2)], rows_v)

        @pl.when(c == 1)
        def _():
            pltpu.sync_copy(md_hbm.at[pl.ds(base, C2)], rows_v)

        pltpu.sync_copy(rows_v, acc.at[idx_v], add=True)
        return 0

    lax.fori_loop(0, EPS // C2, body, 0)
    plsc.subcore_barrier()

    def dump(j, _):
        @pl.when(j % NS == s)
        def _():
            r0 = j * ZC
            pltpu.sync_copy(acc.at[pl.ds(r0, ZC)], rows_v)
            pltpu.sync_copy(rows_v, out_hbm.at[pl.ds(c * N_DST + r0, ZC)])
        return 0

    lax.fori_loop(0, NZCH, dump, 0)


def _sc_scatter(mn, md, dst, zeros):
    mesh = plsc.VectorSubcoreMesh(core_axis_name="c", subcore_axis_name="s")
    f = functools.partial(
        pl.kernel,
        mesh=mesh,
        out_type=jax.ShapeDtypeStruct((NC * N_DST, D), jnp.float32),
        scratch_types=[
            pltpu.VMEM((C2,), jnp.int32),
            pltpu.VMEM((C2, D), jnp.float32),
            pltpu.VMEM_SHARED((N_DST, D), jnp.float32),
        ],
    )(_sc_scatter_body)
    return f(mn, md, dst, zeros)


def _update_body(p0_ref, p1_ref, w0_ref, b0_ref, w1_ref, b1_ref, rep_ref,
                 out_ref):
    num = p0_ref[...]
    den = p1_ref[:, :HEADS]
    denw = jnp.dot(den, rep_ref[...],
                   preferred_element_type=jnp.float32) + 1e-16
    h = jnp.maximum(num / denw, 0.0)
    y0 = jnp.maximum(
        jnp.dot(h, w0_ref[...], preferred_element_type=jnp.float32)
        + b0_ref[...] + h, 0.0)
    out_ref[...] = jnp.maximum(
        jnp.dot(y0, w1_ref[...], preferred_element_type=jnp.float32)
        + b1_ref[...] + y0, 0.0)


def _tc_update(p, w0, b0, w1, b1, rep):
    blk = 1000
    grid = (N_DST // blk,)
    return pl.pallas_call(
        _update_body,
        grid=grid,
        in_specs=[
            pl.BlockSpec((blk, D), lambda i: (i, 0)),
            pl.BlockSpec((blk, D), lambda i: (i + N_DST // blk, 0)),
            pl.BlockSpec((D, D), lambda i: (0, 0)),
            pl.BlockSpec((1, D), lambda i: (0, 0)),
            pl.BlockSpec((D, D), lambda i: (0, 0)),
            pl.BlockSpec((1, D), lambda i: (0, 0)),
            pl.BlockSpec((HEADS, D), lambda i: (0, 0)),
        ],
        out_specs=pl.BlockSpec((blk, D), lambda i: (i, 0)),
        out_shape=jax.ShapeDtypeStruct((N_DST, D), jnp.float32),
    )(p, p, w0, b0, w1, b1, rep)


def kernel(x_src, x_dst, edge_attr, edge_index, q, kW0, kb0, kW1, kb1,
           vW0, vb0, vW1, vb1, oW0, ob0, oW1, ob1):
    f32 = jnp.float32
    w_src = jnp.concatenate([kW0[:D], vW0[:D]], axis=1)
    w_dst = jnp.concatenate([kW0[D:2 * D], vW0[D:2 * D]], axis=1)
    b_dst = jnp.concatenate([kb0, vb0]).reshape(1, TW)
    wke = kW0[2 * D:]
    wve = vW0[2 * D:]
    scale = np.float32(1.0 / np.sqrt(float(D_HEAD)))
    qflat = q.reshape(D)
    sel = (jnp.arange(D)[:, None] // D_HEAD
           == jnp.arange(HEADS)[None, :]).astype(f32)
    qk = scale * ((kW1 + jnp.eye(D, dtype=f32)) @ (qflat[:, None] * sel))
    ck = (scale * jnp.sum((kb1 * qflat).reshape(HEADS, D_HEAD), axis=1)
          ).reshape(1, HEADS)
    rep = sel.T

    src = edge_index[0].astype(jnp.int32)
    dst = edge_index[1].astype(jnp.int32)

    ts, td = _tc_tables(x_src, x_dst, w_src, w_dst, b_dst)
    gs, gd = _sc_gather(ts, td, src, dst)
    mn, md = _tc_edges(gs, gd, edge_attr, wke, wve, qk, ck, vW1,
                       vb1.reshape(1, D), rep)
    p = _sc_scatter(mn, md, dst, jnp.zeros((ZC, D), f32))
    return _tc_update(p, oW0, ob0.reshape(1, D), oW1, ob1.reshape(1, D), rep)


# speedup vs baseline: 4.4877x; 1.3074x over previous
import functools

import jax
import jax.numpy as jnp
import numpy as np
from jax import lax
from jax.experimental import pallas as pl
from jax.experimental.pallas import tpu as pltpu
from jax.experimental.pallas import tpu_sc as plsc

N_SRC = 10000
N_DST = 10000
E = 320000
D = 128
D_EDGE = 16
HEADS = 8
D_HEAD = 16
TW = 2 * D

NC, NS = 2, 16
NW = NC * NS
EPW = E // NW
EPS = E // NS

C1 = 80
C2 = 80
ZC = 80
NZCH = N_DST // ZC


def _tables_body(xs_ref, xd_ref, ws_ref, wd_ref, bd_ref, ts_ref, td_ref):
    ts_ref[...] = jnp.dot(xs_ref[...], ws_ref[...],
                          preferred_element_type=jnp.float32)
    td_ref[...] = jnp.dot(xd_ref[...], wd_ref[...],
                          preferred_element_type=jnp.float32) + bd_ref[...]


def _tc_tables(x_src, x_dst, w_src, w_dst, b_dst):
    blk = 1000
    grid = (N_SRC // blk,)
    return pl.pallas_call(
        _tables_body,
        grid=grid,
        in_specs=[
            pl.BlockSpec((blk, D), lambda i: (i, 0)),
            pl.BlockSpec((blk, D), lambda i: (i, 0)),
            pl.BlockSpec((D, TW), lambda i: (0, 0)),
            pl.BlockSpec((D, TW), lambda i: (0, 0)),
            pl.BlockSpec((1, TW), lambda i: (0, 0)),
        ],
        out_specs=[
            pl.BlockSpec((blk, TW), lambda i: (i, 0)),
            pl.BlockSpec((blk, TW), lambda i: (i, 0)),
        ],
        out_shape=[
            jax.ShapeDtypeStruct((N_SRC, TW), jnp.float32),
            jax.ShapeDtypeStruct((N_DST, TW), jnp.float32),
        ],
    )(x_src, x_dst, w_src, w_dst, b_dst)


def _gather_start(i, wid, src_hbm, dst_hbm, ts_hbm, td_hbm,
                  isv, idv, bs, bd, semg):
    base = wid * EPW + i * C1
    pltpu.sync_copy(src_hbm.at[pl.ds(base, C1)], isv)
    pltpu.sync_copy(dst_hbm.at[pl.ds(base, C1)], idv)
    pltpu.async_copy(ts_hbm.at[isv], bs, semg)
    pltpu.async_copy(td_hbm.at[idv], bd, semg)


def _gather_finish(i, wid, ts_hbm, td_hbm, g_hbm, isv, idv, bs, bd,
                   semg, semw):
    pltpu.make_async_copy(ts_hbm.at[isv], bs, semg).wait()
    pltpu.make_async_copy(td_hbm.at[idv], bd, semg).wait()

    def add_row(r, _):
        for k in range(TW // 16):
            sl = pl.ds(k * 16, 16)
            bs[r, sl] = bs[r, sl] + bd[r, sl]
        return 0

    lax.fori_loop(0, C1, add_row, 0)
    pltpu.async_copy(bs, g_hbm.at[pl.ds(wid * EPW + i * C1, C1)], semw)


def _sc_gather_body(ts_hbm, td_hbm, src_hbm, dst_hbm, g_hbm,
                    is0, id0, bs0, bd0, is1, id1, bs1, bd1,
                    semg0, semg1, semw0, semw1):
    wid = lax.axis_index("s") * NC + lax.axis_index("c")
    nch = EPW // C1

    def work(i, isA, idA, bsA, bdA, semgA, semwA,
             isB, idB, bsB, bdB, semgB, semwB):
        @pl.when(i >= 2)
        def _():
            pltpu.make_async_copy(
                bsA, g_hbm.at[pl.ds(wid * EPW + (i - 2) * C1, C1)],
                semwA).wait()

        _gather_start(i, wid, src_hbm, dst_hbm, ts_hbm, td_hbm,
                      isA, idA, bsA, bdA, semgA)

        @pl.when(i >= 1)
        def _():
            _gather_finish(i - 1, wid, ts_hbm, td_hbm, g_hbm,
                           isB, idB, bsB, bdB, semgB, semwB)

    def body(i, _):
        @pl.when(i % 2 == 0)
        def _():
            work(i, is0, id0, bs0, bd0, semg0, semw0,
                 is1, id1, bs1, bd1, semg1, semw1)

        @pl.when(i % 2 == 1)
        def _():
            work(i, is1, id1, bs1, bd1, semg1, semw1,
                 is0, id0, bs0, bd0, semg0, semw0)

        return 0

    lax.fori_loop(0, nch, body, 0)
    _gather_finish(nch - 1, wid, ts_hbm, td_hbm, g_hbm,
                   is0, id0, bs0, bd0, semg0, semw0)
    pltpu.make_async_copy(
        bs1, g_hbm.at[pl.ds(wid * EPW + (nch - 2) * C1, C1)], semw1).wait()
    pltpu.make_async_copy(
        bs0, g_hbm.at[pl.ds(wid * EPW + (nch - 1) * C1, C1)], semw0).wait()


def _sc_gather(ts, td, src, dst):
    mesh = plsc.VectorSubcoreMesh(core_axis_name="c", subcore_axis_name="s")
    f = functools.partial(
        pl.kernel,
        mesh=mesh,
        out_type=jax.ShapeDtypeStruct((E, TW), jnp.float32),
        scratch_types=[
            pltpu.VMEM((C1,), jnp.int32),
            pltpu.VMEM((C1,), jnp.int32),
            pltpu.VMEM((C1, TW), jnp.float32),
            pltpu.VMEM((C1, TW), jnp.float32),
            pltpu.VMEM((C1,), jnp.int32),
            pltpu.VMEM((C1,), jnp.int32),
            pltpu.VMEM((C1, TW), jnp.float32),
            pltpu.VMEM((C1, TW), jnp.float32),
            pltpu.SemaphoreType.DMA,
            pltpu.SemaphoreType.DMA,
            pltpu.SemaphoreType.DMA,
            pltpu.SemaphoreType.DMA,
        ],
    )(_sc_gather_body)
    return f(ts, td, src, dst)


def _edge_body(g_ref, ea_ref, wke_ref, wve_ref, qk_ref, ck_ref,
               wv1_ref, bv1_ref, rep_ref, mn_ref, ex_ref):
    ea = ea_ref[...]
    h0k = jnp.maximum(
        g_ref[:, :D]
        + jnp.dot(ea, wke_ref[...], preferred_element_type=jnp.float32), 0.0)
    coef = jnp.dot(h0k, qk_ref[...],
                   preferred_element_type=jnp.float32) + ck_ref[...]
    ex = jnp.exp(coef)
    h0v = jnp.maximum(
        g_ref[:, D:]
        + jnp.dot(ea, wve_ref[...], preferred_element_type=jnp.float32), 0.0)
    v1 = jnp.dot(h0v, wv1_ref[...],
                 preferred_element_type=jnp.float32) + bv1_ref[...] + h0v
    exw = jnp.dot(ex, rep_ref[...],
                  preferred_element_type=jnp.float32)
    mn_ref[...] = exw * v1
    ex_ref[...] = ex


def _tc_edges(g, ea, wke, wve, qk, ck, wv1, bv1, rep):
    blk = 1000
    grid = (E // blk,)
    return pl.pallas_call(
        _edge_body,
        grid=grid,
        in_specs=[
            pl.BlockSpec((blk, TW), lambda i: (i, 0)),
            pl.BlockSpec((blk, D_EDGE), lambda i: (i, 0)),
            pl.BlockSpec((D_EDGE, D), lambda i: (0, 0)),
            pl.BlockSpec((D_EDGE, D), lambda i: (0, 0)),
            pl.BlockSpec((D, HEADS), lambda i: (0, 0)),
            pl.BlockSpec((1, HEADS), lambda i: (0, 0)),
            pl.BlockSpec((D, D), lambda i: (0, 0)),
            pl.BlockSpec((1, D), lambda i: (0, 0)),
            pl.BlockSpec((HEADS, D), lambda i: (0, 0)),
        ],
        out_specs=[
            pl.BlockSpec((blk, D), lambda i: (i, 0)),
            pl.BlockSpec((blk, HEADS), lambda i: (i, 0)),
        ],
        out_shape=[
            jax.ShapeDtypeStruct((E, D), jnp.float32),
            jax.ShapeDtypeStruct((E, HEADS), jnp.float32),
        ],
    )(g, ea, wke, wve, qk, ck, wv1, bv1, rep)


def _scatter_start(i, s, c, dst_hbm, mn_hbm, ex_hbm, idx, rows, exb, semm):
    base = s * EPS + i * C2
    pltpu.sync_copy(dst_hbm.at[pl.ds(base, C2)], idx)

    @pl.when(c == 0)
    def _():
        pltpu.async_copy(mn_hbm.at[pl.ds(base, C2)], rows, semm)

    @pl.when(c == 1)
    def _():
        pltpu.async_copy(ex_hbm.at[pl.ds(base * HEADS, C2 * HEADS)],
                         exb.at[pl.ds(0, C2 * HEADS)], semm)


def _scatter_finish(i, s, c, mn_hbm, ex_hbm, acc, idx, rows, exb, semm):
    base = s * EPS + i * C2

    @pl.when(c == 0)
    def _():
        pltpu.make_async_copy(mn_hbm.at[pl.ds(base, C2)], rows, semm).wait()

    @pl.when(c == 1)
    def _():
        pltpu.make_async_copy(
            ex_hbm.at[pl.ds(base * HEADS, C2 * HEADS)],
            exb.at[pl.ds(0, C2 * HEADS)], semm).wait()
        low = lax.iota(jnp.int32, 16) < HEADS

        def expand(r, _):
            vec = jnp.where(low, exb[pl.ds(r * HEADS, 16)], 0.0)
            rows[r, pl.ds(0, 16)] = vec
            return 0

        lax.fori_loop(0, C2, expand, 0)

    pltpu.sync_copy(rows, acc.at[idx], add=True)


def _sc_scatter_body(mn_hbm, ex_hbm, dst_hbm, z_hbm, out_hbm,
                     idx0, rows0, exb0, idx1, rows1, exb1,
                     semm0, semm1, acc):
    c = lax.axis_index("c")
    s = lax.axis_index("s")
    nch = EPS // C2

    pltpu.sync_copy(z_hbm, rows0)
    pltpu.sync_copy(z_hbm, rows1)

    def zinit(j, _):
        @pl.when(j % NS == s)
        def _():
            pltpu.sync_copy(rows0, acc.at[pl.ds(j * ZC, ZC)])
        return 0

    lax.fori_loop(0, NZCH, zinit, 0)
    plsc.subcore_barrier()

    def body(i, _):
        @pl.when(i % 2 == 0)
        def _():
            _scatter_start(i, s, c, dst_hbm, mn_hbm, ex_hbm,
                           idx0, rows0, exb0, semm0)

            @pl.when(i >= 1)
            def _():
                _scatter_finish(i - 1, s, c, mn_hbm, ex_hbm, acc,
                                idx1, rows1, exb1, semm1)

        @pl.when(i % 2 == 1)
        def _():
            _scatter_start(i, s, c, dst_hbm, mn_hbm, ex_hbm,
                           idx1, rows1, exb1, semm1)

            @pl.when(i >= 1)
            def _():
                _scatter_finish(i - 1, s, c, mn_hbm, ex_hbm, acc,
                                idx0, rows0, exb0, semm0)

        return 0

    lax.fori_loop(0, nch, body, 0)
    _scatter_finish(nch - 1, s, c, mn_hbm, ex_hbm, acc,
                    idx1, rows1, exb1, semm1)
    plsc.subcore_barrier()

    def dump(j, _):
        @pl.when(j % NS == s)
        def _():
            r0 = j * ZC
            pltpu.sync_copy(acc.at[pl.ds(r0, ZC)], rows0)
            pltpu.sync_copy(rows0, out_hbm.at[pl.ds(c * N_DST + r0, ZC)])
        return 0

    lax.fori_loop(0, NZCH, dump, 0)


def _sc_scatter(mn, ex_flat, dst, zeros):
    mesh = plsc.VectorSubcoreMesh(core_axis_name="c", subcore_axis_name="s")
    f = functools.partial(
        pl.kernel,
        mesh=mesh,
        out_type=jax.ShapeDtypeStruct((NC * N_DST, D), jnp.float32),
        scratch_types=[
            pltpu.VMEM((C2,), jnp.int32),
            pltpu.VMEM((C2, D), jnp.float32),
            pltpu.VMEM((C2 * HEADS + 16,), jnp.float32),
            pltpu.VMEM((C2,), jnp.int32),
            pltpu.VMEM((C2, D), jnp.float32),
            pltpu.VMEM((C2 * HEADS + 16,), jnp.float32),
            pltpu.SemaphoreType.DMA,
            pltpu.SemaphoreType.DMA,
            pltpu.VMEM_SHARED((N_DST, D), jnp.float32),
        ],
    )(_sc_scatter_body)
    return f(mn, ex_flat, dst, zeros)


def _update_body(p0_ref, p1_ref, w0_ref, b0_ref, w1_ref, b1_ref, rep_ref,
                 out_ref):
    num = p0_ref[...]
    den = p1_ref[:, :HEADS]
    denw = jnp.dot(den, rep_ref[...],
                   preferred_element_type=jnp.float32) + 1e-16
    h = jnp.maximum(num / denw, 0.0)
    y0 = jnp.maximum(
        jnp.dot(h, w0_ref[...], preferred_element_type=jnp.float32)
        + b0_ref[...] + h, 0.0)
    out_ref[...] = jnp.maximum(
        jnp.dot(y0, w1_ref[...], preferred_element_type=jnp.float32)
        + b1_ref[...] + y0, 0.0)


def _tc_update(p, w0, b0, w1, b1, rep):
    blk = 1000
    grid = (N_DST // blk,)
    return pl.pallas_call(
        _update_body,
        grid=grid,
        in_specs=[
            pl.BlockSpec((blk, D), lambda i: (i, 0)),
            pl.BlockSpec((blk, D), lambda i: (i + N_DST // blk, 0)),
            pl.BlockSpec((D, D), lambda i: (0, 0)),
            pl.BlockSpec((1, D), lambda i: (0, 0)),
            pl.BlockSpec((D, D), lambda i: (0, 0)),
            pl.BlockSpec((1, D), lambda i: (0, 0)),
            pl.BlockSpec((HEADS, D), lambda i: (0, 0)),
        ],
        out_specs=pl.BlockSpec((blk, D), lambda i: (i, 0)),
        out_shape=jax.ShapeDtypeStruct((N_DST, D), jnp.float32),
    )(p, p, w0, b0, w1, b1, rep)


def kernel(x_src, x_dst, edge_attr, edge_index, q, kW0, kb0, kW1, kb1,
           vW0, vb0, vW1, vb1, oW0, ob0, oW1, ob1):
    f32 = jnp.float32
    w_src = jnp.concatenate([kW0[:D], vW0[:D]], axis=1)
    w_dst = jnp.concatenate([kW0[D:2 * D], vW0[D:2 * D]], axis=1)
    b_dst = jnp.concatenate([kb0, vb0]).reshape(1, TW)
    wke = kW0[2 * D:]
    wve = vW0[2 * D:]
    scale = np.float32(1.0 / np.sqrt(float(D_HEAD)))
    qflat = q.reshape(D)
    sel = (jnp.arange(D)[:, None] // D_HEAD
           == jnp.arange(HEADS)[None, :]).astype(f32)
    qk = scale * ((kW1 + jnp.eye(D, dtype=f32)) @ (qflat[:, None] * sel))
    ck = (scale * jnp.sum((kb1 * qflat).reshape(HEADS, D_HEAD), axis=1)
          ).reshape(1, HEADS)
    rep = sel.T

    src = edge_index[0].astype(jnp.int32)
    dst = edge_index[1].astype(jnp.int32)

    ts, td = _tc_tables(x_src, x_dst, w_src, w_dst, b_dst)
    g = _sc_gather(ts, td, src, dst)
    mn, ex8 = _tc_edges(g, edge_attr, wke, wve, qk, ck, vW1,
                        vb1.reshape(1, D), rep)
    p = _sc_scatter(mn, ex8.reshape(E * HEADS), dst, jnp.zeros((ZC, D), f32))
    return _tc_update(p, oW0, ob0.reshape(1, D), oW1, ob1.reshape(1, D), rep)


# speedup vs baseline: 5.2148x; 1.1620x over previous
import functools

import jax
import jax.numpy as jnp
import numpy as np
from jax import lax
from jax.experimental import pallas as pl
from jax.experimental.pallas import tpu as pltpu
from jax.experimental.pallas import tpu_sc as plsc

N_SRC = 10000
N_DST = 10000
E = 320000
D = 128
D_EDGE = 16
HEADS = 8
D_HEAD = 16
TW = 2 * D

NC, NS = 2, 16
NW = NC * NS
EH = E // 2
EPW = EH // NW
EPS = EH // NS

C1 = 40
C2 = 80
ZC = 80
NZCH = N_DST // ZC


def _tables_body(xs_ref, xd_ref, ws_ref, wd_ref, bd_ref, ts_ref, td_ref):
    ts_ref[...] = jnp.dot(xs_ref[...], ws_ref[...],
                          preferred_element_type=jnp.float32)
    td_ref[...] = jnp.dot(xd_ref[...], wd_ref[...],
                          preferred_element_type=jnp.float32) + bd_ref[...]


def _tc_tables(x_src, x_dst, w_src, w_dst, b_dst):
    blk = 1000
    grid = (N_SRC // blk,)
    return pl.pallas_call(
        _tables_body,
        grid=grid,
        in_specs=[
            pl.BlockSpec((blk, D), lambda i: (i, 0)),
            pl.BlockSpec((blk, D), lambda i: (i, 0)),
            pl.BlockSpec((D, TW), lambda i: (0, 0)),
            pl.BlockSpec((D, TW), lambda i: (0, 0)),
            pl.BlockSpec((1, TW), lambda i: (0, 0)),
        ],
        out_specs=[
            pl.BlockSpec((blk, TW), lambda i: (i, 0)),
            pl.BlockSpec((blk, TW), lambda i: (i, 0)),
        ],
        out_shape=[
            jax.ShapeDtypeStruct((N_SRC, TW), jnp.float32),
            jax.ShapeDtypeStruct((N_DST, TW), jnp.float32),
        ],
    )(x_src, x_dst, w_src, w_dst, b_dst)


def _gather_start(i, wid, src_hbm, dst_hbm, ts_hbm, td_hbm,
                  isv, idv, bs, bd, semg):
    base = wid * EPW + i * C1
    pltpu.sync_copy(src_hbm.at[pl.ds(base, C1)], isv)
    pltpu.sync_copy(dst_hbm.at[pl.ds(base, C1)], idv)
    pltpu.async_copy(ts_hbm.at[isv], bs, semg)
    pltpu.async_copy(td_hbm.at[idv], bd, semg)


def _gather_finish(i, wid, ts_hbm, td_hbm, g_hbm, isv, idv, bs, bd,
                   semg, semw):
    pltpu.make_async_copy(ts_hbm.at[isv], bs, semg).wait()
    pltpu.make_async_copy(td_hbm.at[idv], bd, semg).wait()

    def add_row(r, _):
        for k in range(TW // 16):
            sl = pl.ds(k * 16, 16)
            bs[r, sl] = bs[r, sl] + bd[r, sl]
        return 0

    lax.fori_loop(0, C1, add_row, 0)
    pltpu.async_copy(bs, g_hbm.at[pl.ds(wid * EPW + i * C1, C1)], semw)


def _sc_gather_body(ts_hbm, td_hbm, src_hbm, dst_hbm, g_hbm,
                    is0, id0, bs0, bd0, is1, id1, bs1, bd1,
                    semg0, semg1, semw0, semw1):
    wid = lax.axis_index("s") * NC + lax.axis_index("c")
    nch = EPW // C1

    def work(i, isA, idA, bsA, bdA, semgA, semwA,
             isB, idB, bsB, bdB, semgB, semwB):
        @pl.when(i >= 2)
        def _():
            pltpu.make_async_copy(
                bsA, g_hbm.at[pl.ds(wid * EPW + (i - 2) * C1, C1)],
                semwA).wait()

        _gather_start(i, wid, src_hbm, dst_hbm, ts_hbm, td_hbm,
                      isA, idA, bsA, bdA, semgA)

        @pl.when(i >= 1)
        def _():
            _gather_finish(i - 1, wid, ts_hbm, td_hbm, g_hbm,
                           isB, idB, bsB, bdB, semgB, semwB)

    def body(i, _):
        @pl.when(i % 2 == 0)
        def _():
            work(i, is0, id0, bs0, bd0, semg0, semw0,
                 is1, id1, bs1, bd1, semg1, semw1)

        @pl.when(i % 2 == 1)
        def _():
            work(i, is1, id1, bs1, bd1, semg1, semw1,
                 is0, id0, bs0, bd0, semg0, semw0)

        return 0

    lax.fori_loop(0, nch, body, 0)
    _gather_finish(nch - 1, wid, ts_hbm, td_hbm, g_hbm,
                   is0, id0, bs0, bd0, semg0, semw0)
    pltpu.make_async_copy(
        bs1, g_hbm.at[pl.ds(wid * EPW + (nch - 2) * C1, C1)], semw1).wait()
    pltpu.make_async_copy(
        bs0, g_hbm.at[pl.ds(wid * EPW + (nch - 1) * C1, C1)], semw0).wait()


def _sc_gather(ts, td, src, dst):
    mesh = plsc.VectorSubcoreMesh(core_axis_name="c", subcore_axis_name="s")
    f = functools.partial(
        pl.kernel,
        mesh=mesh,
        out_type=jax.ShapeDtypeStruct((EH, TW), jnp.float32),
        scratch_types=[
            pltpu.VMEM((C1,), jnp.int32),
            pltpu.VMEM((C1,), jnp.int32),
            pltpu.VMEM((C1, TW), jnp.float32),
            pltpu.VMEM((C1, TW), jnp.float32),
            pltpu.VMEM((C1,), jnp.int32),
            pltpu.VMEM((C1,), jnp.int32),
            pltpu.VMEM((C1, TW), jnp.float32),
            pltpu.VMEM((C1, TW), jnp.float32),
            pltpu.SemaphoreType.DMA,
            pltpu.SemaphoreType.DMA,
            pltpu.SemaphoreType.DMA,
            pltpu.SemaphoreType.DMA,
        ],
    )(_sc_gather_body)
    return f(ts, td, src, dst)


def _edge_body(g_ref, ea_ref, wke_ref, wve_ref, qk_ref, ck_ref,
               wv1_ref, bv1_ref, rep_ref, mn_ref, ex_ref):
    ea = ea_ref[...]
    h0k = jnp.maximum(
        g_ref[:, :D]
        + jnp.dot(ea, wke_ref[...], preferred_element_type=jnp.float32), 0.0)
    coef = jnp.dot(h0k, qk_ref[...],
                   preferred_element_type=jnp.float32) + ck_ref[...]
    ex = jnp.exp(coef)
    h0v = jnp.maximum(
        g_ref[:, D:]
        + jnp.dot(ea, wve_ref[...], preferred_element_type=jnp.float32), 0.0)
    v1 = jnp.dot(h0v, wv1_ref[...],
                 preferred_element_type=jnp.float32) + bv1_ref[...] + h0v
    exw = jnp.dot(ex, rep_ref[...],
                  preferred_element_type=jnp.float32)
    mn_ref[...] = exw * v1
    ex_ref[...] = ex


def _tc_edges(g, ea, wke, wve, qk, ck, wv1, bv1, rep):
    blk = 1000
    grid = (EH // blk,)
    return pl.pallas_call(
        _edge_body,
        grid=grid,
        in_specs=[
            pl.BlockSpec((blk, TW), lambda i: (i, 0)),
            pl.BlockSpec((blk, D_EDGE), lambda i: (i, 0)),
            pl.BlockSpec((D_EDGE, D), lambda i: (0, 0)),
            pl.BlockSpec((D_EDGE, D), lambda i: (0, 0)),
            pl.BlockSpec((D, HEADS), lambda i: (0, 0)),
            pl.BlockSpec((1, HEADS), lambda i: (0, 0)),
            pl.BlockSpec((D, D), lambda i: (0, 0)),
            pl.BlockSpec((1, D), lambda i: (0, 0)),
            pl.BlockSpec((HEADS, D), lambda i: (0, 0)),
        ],
        out_specs=[
            pl.BlockSpec((blk, D), lambda i: (i, 0)),
            pl.BlockSpec((blk, HEADS), lambda i: (i, 0)),
        ],
        out_shape=[
            jax.ShapeDtypeStruct((EH, D), jnp.float32),
            jax.ShapeDtypeStruct((EH, HEADS), jnp.float32),
        ],
    )(g, ea, wke, wve, qk, ck, wv1, bv1, rep)


def _scatter_start(i, s, c, dst_hbm, mn_hbm, ex_hbm, idx, rows, exb, semm):
    base = s * EPS + i * C2
    pltpu.sync_copy(dst_hbm.at[pl.ds(base, C2)], idx)

    @pl.when(c == 0)
    def _():
        pltpu.async_copy(mn_hbm.at[pl.ds(base, C2)], rows, semm)

    @pl.when(c == 1)
    def _():
        pltpu.async_copy(ex_hbm.at[pl.ds(base * HEADS, C2 * HEADS)],
                         exb.at[pl.ds(0, C2 * HEADS)], semm)


def _scatter_finish(i, s, c, mn_hbm, ex_hbm, acc, idx, rows, exb, semm):
    base = s * EPS + i * C2

    @pl.when(c == 0)
    def _():
        pltpu.make_async_copy(mn_hbm.at[pl.ds(base, C2)], rows, semm).wait()

    @pl.when(c == 1)
    def _():
        pltpu.make_async_copy(
            ex_hbm.at[pl.ds(base * HEADS, C2 * HEADS)],
            exb.at[pl.ds(0, C2 * HEADS)], semm).wait()
        low = lax.iota(jnp.int32, 16) < HEADS

        def expand(r, _):
            vec = jnp.where(low, exb[pl.ds(r * HEADS, 16)], 0.0)
            rows[r, pl.ds(0, 16)] = vec
            return 0

        lax.fori_loop(0, C2, expand, 0)

    pltpu.sync_copy(rows, acc.at[idx], add=True)


def _sc_scatter_body(mn_hbm, ex_hbm, dst_hbm, z_hbm, out_hbm,
                     idx0, rows0, exb0, idx1, rows1, exb1,
                     semm0, semm1, acc):
    c = lax.axis_index("c")
    s = lax.axis_index("s")
    nch = EPS // C2

    pltpu.sync_copy(z_hbm, rows0)
    pltpu.sync_copy(z_hbm, rows1)

    def zinit(j, _):
        @pl.when(j % NS == s)
        def _():
            pltpu.sync_copy(rows0, acc.at[pl.ds(j * ZC, ZC)])
        return 0

    lax.fori_loop(0, NZCH, zinit, 0)
    plsc.subcore_barrier()

    def body(i, _):
        @pl.when(i % 2 == 0)
        def _():
            _scatter_start(i, s, c, dst_hbm, mn_hbm, ex_hbm,
                           idx0, rows0, exb0, semm0)

            @pl.when(i >= 1)
            def _():
                _scatter_finish(i - 1, s, c, mn_hbm, ex_hbm, acc,
                                idx1, rows1, exb1, semm1)

        @pl.when(i % 2 == 1)
        def _():
            _scatter_start(i, s, c, dst_hbm, mn_hbm, ex_hbm,
                           idx1, rows1, exb1, semm1)

            @pl.when(i >= 1)
            def _():
                _scatter_finish(i - 1, s, c, mn_hbm, ex_hbm, acc,
                                idx0, rows0, exb0, semm0)

        return 0

    lax.fori_loop(0, nch, body, 0)
    if (nch - 1) % 2 == 0:
        _scatter_finish(nch - 1, s, c, mn_hbm, ex_hbm, acc,
                        idx0, rows0, exb0, semm0)
    else:
        _scatter_finish(nch - 1, s, c, mn_hbm, ex_hbm, acc,
                        idx1, rows1, exb1, semm1)
    plsc.subcore_barrier()

    def dump(j, _):
        @pl.when(j % NS == s)
        def _():
            r0 = j * ZC
            pltpu.sync_copy(acc.at[pl.ds(r0, ZC)], rows0)
            pltpu.sync_copy(rows0, out_hbm.at[pl.ds(c * N_DST + r0, ZC)])
        return 0

    lax.fori_loop(0, NZCH, dump, 0)


def _sc_scatter(mn, ex_flat, dst, zeros):
    mesh = plsc.VectorSubcoreMesh(core_axis_name="c", subcore_axis_name="s")
    f = functools.partial(
        pl.kernel,
        mesh=mesh,
        out_type=jax.ShapeDtypeStruct((NC * N_DST, D), jnp.float32),
        scratch_types=[
            pltpu.VMEM((C2,), jnp.int32),
            pltpu.VMEM((C2, D), jnp.float32),
            pltpu.VMEM((C2 * HEADS + 16,), jnp.float32),
            pltpu.VMEM((C2,), jnp.int32),
            pltpu.VMEM((C2, D), jnp.float32),
            pltpu.VMEM((C2 * HEADS + 16,), jnp.float32),
            pltpu.SemaphoreType.DMA,
            pltpu.SemaphoreType.DMA,
            pltpu.VMEM_SHARED((N_DST, D), jnp.float32),
        ],
    )(_sc_scatter_body)
    return f(mn, ex_flat, dst, zeros)


def _update_body(p1n_ref, p1d_ref, p2n_ref, p2d_ref, w0_ref, b0_ref,
                 w1_ref, b1_ref, rep_ref, out_ref):
    num = p1n_ref[...] + p2n_ref[...]
    den = p1d_ref[:, :HEADS] + p2d_ref[:, :HEADS]
    denw = jnp.dot(den, rep_ref[...],
                   preferred_element_type=jnp.float32) + 1e-16
    h = jnp.maximum(num / denw, 0.0)
    y0 = jnp.maximum(
        jnp.dot(h, w0_ref[...], preferred_element_type=jnp.float32)
        + b0_ref[...] + h, 0.0)
    out_ref[...] = jnp.maximum(
        jnp.dot(y0, w1_ref[...], preferred_element_type=jnp.float32)
        + b1_ref[...] + y0, 0.0)


def _tc_update(p1, p2, w0, b0, w1, b1, rep):
    blk = 1000
    grid = (N_DST // blk,)
    nb = N_DST // blk
    return pl.pallas_call(
        _update_body,
        grid=grid,
        in_specs=[
            pl.BlockSpec((blk, D), lambda i: (i, 0)),
            pl.BlockSpec((blk, D), lambda i: (i + nb, 0)),
            pl.BlockSpec((blk, D), lambda i: (i, 0)),
            pl.BlockSpec((blk, D), lambda i: (i + nb, 0)),
            pl.BlockSpec((D, D), lambda i: (0, 0)),
            pl.BlockSpec((1, D), lambda i: (0, 0)),
            pl.BlockSpec((D, D), lambda i: (0, 0)),
            pl.BlockSpec((1, D), lambda i: (0, 0)),
            pl.BlockSpec((HEADS, D), lambda i: (0, 0)),
        ],
        out_specs=pl.BlockSpec((blk, D), lambda i: (i, 0)),
        out_shape=jax.ShapeDtypeStruct((N_DST, D), jnp.float32),
    )(p1, p1, p2, p2, w0, b0, w1, b1, rep)


def kernel(x_src, x_dst, edge_attr, edge_index, q, kW0, kb0, kW1, kb1,
           vW0, vb0, vW1, vb1, oW0, ob0, oW1, ob1):
    f32 = jnp.float32
    w_src = jnp.concatenate([kW0[:D], vW0[:D]], axis=1)
    w_dst = jnp.concatenate([kW0[D:2 * D], vW0[D:2 * D]], axis=1)
    b_dst = jnp.concatenate([kb0, vb0]).reshape(1, TW)
    wke = kW0[2 * D:]
    wve = vW0[2 * D:]
    scale = np.float32(1.0 / np.sqrt(float(D_HEAD)))
    qflat = q.reshape(D)
    sel = (jnp.arange(D)[:, None] // D_HEAD
           == jnp.arange(HEADS)[None, :]).astype(f32)
    qk = scale * ((kW1 + jnp.eye(D, dtype=f32)) @ (qflat[:, None] * sel))
    ck = (scale * jnp.sum((kb1 * qflat).reshape(HEADS, D_HEAD), axis=1)
          ).reshape(1, HEADS)
    rep = sel.T

    src = edge_index[0].astype(jnp.int32)
    dst = edge_index[1].astype(jnp.int32)

    ts, td = _tc_tables(x_src, x_dst, w_src, w_dst, b_dst)
    zeros = jnp.zeros((ZC, D), f32)
    ps = []
    for h in range(2):
        sl = slice(h * EH, (h + 1) * EH)
        g = _sc_gather(ts, td, src[sl], dst[sl])
        mn, ex8 = _tc_edges(g, edge_attr[sl], wke, wve, qk, ck, vW1,
                            vb1.reshape(1, D), rep)
        ps.append(_sc_scatter(mn, ex8.reshape(EH * HEADS), dst[sl], zeros))
    return _tc_update(ps[0], ps[1], oW0, ob0.reshape(1, D),
                      oW1, ob1.reshape(1, D), rep)


# speedup vs baseline: 5.2589x; 1.0085x over previous
import functools

import jax
import jax.numpy as jnp
import numpy as np
from jax import lax
from jax.experimental import pallas as pl
from jax.experimental.pallas import tpu as pltpu
from jax.experimental.pallas import tpu_sc as plsc

N_SRC = 10000
N_DST = 10000
E = 320000
D = 128
D_EDGE = 16
HEADS = 8
D_HEAD = 16
TW = 2 * D

NC, NS = 2, 16
NW = NC * NS
EH = E // 2
EPW = EH // NW
EPS = EH // NS

C1 = 80
T1 = EPW % C1
C2 = 80
ZC = 80
NZCH = N_DST // ZC


def _tables_body(xs_ref, xd_ref, ws_ref, wd_ref, bd_ref, ts_ref, td_ref):
    ts_ref[...] = jnp.dot(xs_ref[...], ws_ref[...],
                          preferred_element_type=jnp.float32)
    td_ref[...] = jnp.dot(xd_ref[...], wd_ref[...],
                          preferred_element_type=jnp.float32) + bd_ref[...]


def _tc_tables(x_src, x_dst, w_src, w_dst, b_dst):
    blk = 1000
    grid = (N_SRC // blk,)
    return pl.pallas_call(
        _tables_body,
        grid=grid,
        in_specs=[
            pl.BlockSpec((blk, D), lambda i: (i, 0)),
            pl.BlockSpec((blk, D), lambda i: (i, 0)),
            pl.BlockSpec((D, TW), lambda i: (0, 0)),
            pl.BlockSpec((D, TW), lambda i: (0, 0)),
            pl.BlockSpec((1, TW), lambda i: (0, 0)),
        ],
        out_specs=[
            pl.BlockSpec((blk, TW), lambda i: (i, 0)),
            pl.BlockSpec((blk, TW), lambda i: (i, 0)),
        ],
        out_shape=[
            jax.ShapeDtypeStruct((N_SRC, TW), jnp.float32),
            jax.ShapeDtypeStruct((N_DST, TW), jnp.float32),
        ],
    )(x_src, x_dst, w_src, w_dst, b_dst)


def _gather_start(i, wid, src_hbm, dst_hbm, ts_hbm, td_hbm,
                  isv, idv, bs, bd, semg):
    base = wid * EPW + i * C1
    pltpu.sync_copy(src_hbm.at[pl.ds(base, C1)], isv)
    pltpu.sync_copy(dst_hbm.at[pl.ds(base, C1)], idv)
    pltpu.async_copy(ts_hbm.at[isv], bs, semg)
    pltpu.async_copy(td_hbm.at[idv], bd, semg)


def _gather_finish(i, wid, ts_hbm, td_hbm, g_hbm, isv, idv, bs, bd,
                   semg, semw):
    pltpu.make_async_copy(ts_hbm.at[isv], bs, semg).wait()
    pltpu.make_async_copy(td_hbm.at[idv], bd, semg).wait()

    def add_row(r, _):
        for k in range(TW // 16):
            sl = pl.ds(k * 16, 16)
            bs[r, sl] = bs[r, sl] + bd[r, sl]
        return 0

    lax.fori_loop(0, C1, add_row, 0)
    pltpu.async_copy(bs, g_hbm.at[pl.ds(wid * EPW + i * C1, C1)], semw)


def _sc_gather_body(ts_hbm, td_hbm, src_hbm, dst_hbm, g_hbm,
                    is0, id0, bs0, bd0, is1, id1, bs1, bd1, ist, idt,
                    semg0, semg1, semw0, semw1):
    wid = lax.axis_index("s") * NC + lax.axis_index("c")
    nch = EPW // C1

    def work(i, isA, idA, bsA, bdA, semgA, semwA,
             isB, idB, bsB, bdB, semgB, semwB):
        @pl.when(i >= 2)
        def _():
            pltpu.make_async_copy(
                bsA, g_hbm.at[pl.ds(wid * EPW + (i - 2) * C1, C1)],
                semwA).wait()

        _gather_start(i, wid, src_hbm, dst_hbm, ts_hbm, td_hbm,
                      isA, idA, bsA, bdA, semgA)

        @pl.when(i >= 1)
        def _():
            _gather_finish(i - 1, wid, ts_hbm, td_hbm, g_hbm,
                           isB, idB, bsB, bdB, semgB, semwB)

    def body(i, _):
        @pl.when(i % 2 == 0)
        def _():
            work(i, is0, id0, bs0, bd0, semg0, semw0,
                 is1, id1, bs1, bd1, semg1, semw1)

        @pl.when(i % 2 == 1)
        def _():
            work(i, is1, id1, bs1, bd1, semg1, semw1,
                 is0, id0, bs0, bd0, semg0, semw0)

        return 0

    lax.fori_loop(0, nch, body, 0)
    if (nch - 1) % 2 == 0:
        _gather_finish(nch - 1, wid, ts_hbm, td_hbm, g_hbm,
                       is0, id0, bs0, bd0, semg0, semw0)
    else:
        _gather_finish(nch - 1, wid, ts_hbm, td_hbm, g_hbm,
                       is1, id1, bs1, bd1, semg1, semw1)
    pltpu.make_async_copy(
        bs1 if (nch - 1) % 2 == 0 else bs0,
        g_hbm.at[pl.ds(wid * EPW + (nch - 2) * C1, C1)],
        semw1 if (nch - 1) % 2 == 0 else semw0).wait()
    pltpu.make_async_copy(
        bs0 if (nch - 1) % 2 == 0 else bs1,
        g_hbm.at[pl.ds(wid * EPW + (nch - 1) * C1, C1)],
        semw0 if (nch - 1) % 2 == 0 else semw1).wait()
    if T1:
        baset = wid * EPW + nch * C1
        pltpu.sync_copy(src_hbm.at[pl.ds(baset, T1)], ist)
        pltpu.sync_copy(dst_hbm.at[pl.ds(baset, T1)], idt)
        pltpu.async_copy(ts_hbm.at[ist], bs0.at[pl.ds(0, T1)], semg0).wait()
        pltpu.async_copy(td_hbm.at[idt], bd0.at[pl.ds(0, T1)], semg0).wait()

        def add_row_t(r, _):
            for k in range(TW // 16):
                sl = pl.ds(k * 16, 16)
                bs0[r, sl] = bs0[r, sl] + bd0[r, sl]
            return 0

        lax.fori_loop(0, T1, add_row_t, 0)
        pltpu.sync_copy(bs0.at[pl.ds(0, T1)], g_hbm.at[pl.ds(baset, T1)])


def _sc_gather(ts, td, src, dst):
    mesh = plsc.VectorSubcoreMesh(core_axis_name="c", subcore_axis_name="s")
    f = functools.partial(
        pl.kernel,
        mesh=mesh,
        out_type=jax.ShapeDtypeStruct((EH, TW), jnp.float32),
        scratch_types=[
            pltpu.VMEM((C1,), jnp.int32),
            pltpu.VMEM((C1,), jnp.int32),
            pltpu.VMEM((C1, TW), jnp.float32),
            pltpu.VMEM((C1, TW), jnp.float32),
            pltpu.VMEM((C1,), jnp.int32),
            pltpu.VMEM((C1,), jnp.int32),
            pltpu.VMEM((C1, TW), jnp.float32),
            pltpu.VMEM((C1, TW), jnp.float32),
            pltpu.VMEM((T1,), jnp.int32),
            pltpu.VMEM((T1,), jnp.int32),
            pltpu.SemaphoreType.DMA,
            pltpu.SemaphoreType.DMA,
            pltpu.SemaphoreType.DMA,
            pltpu.SemaphoreType.DMA,
        ],
    )(_sc_gather_body)
    return f(ts, td, src, dst)


def _edge_body(g_ref, ea_ref, wke_ref, wve_ref, qk_ref, ck_ref,
               wv1_ref, bv1_ref, rep_ref, mn_ref, ex_ref):
    ea = ea_ref[...]
    h0k = jnp.maximum(
        g_ref[:, :D]
        + jnp.dot(ea, wke_ref[...], preferred_element_type=jnp.float32), 0.0)
    coef = jnp.dot(h0k, qk_ref[...],
                   preferred_element_type=jnp.float32) + ck_ref[...]
    ex = jnp.exp(coef)
    h0v = jnp.maximum(
        g_ref[:, D:]
        + jnp.dot(ea, wve_ref[...], preferred_element_type=jnp.float32), 0.0)
    v1 = jnp.dot(h0v, wv1_ref[...],
                 preferred_element_type=jnp.float32) + bv1_ref[...] + h0v
    exw = jnp.dot(ex, rep_ref[...],
                  preferred_element_type=jnp.float32)
    mn_ref[...] = exw * v1
    ex_ref[...] = ex


def _tc_edges(g, ea, wke, wve, qk, ck, wv1, bv1, rep):
    blk = 1000
    grid = (EH // blk,)
    return pl.pallas_call(
        _edge_body,
        grid=grid,
        in_specs=[
            pl.BlockSpec((blk, TW), lambda i: (i, 0)),
            pl.BlockSpec((blk, D_EDGE), lambda i: (i, 0)),
            pl.BlockSpec((D_EDGE, D), lambda i: (0, 0)),
            pl.BlockSpec((D_EDGE, D), lambda i: (0, 0)),
            pl.BlockSpec((D, HEADS), lambda i: (0, 0)),
            pl.BlockSpec((1, HEADS), lambda i: (0, 0)),
            pl.BlockSpec((D, D), lambda i: (0, 0)),
            pl.BlockSpec((1, D), lambda i: (0, 0)),
            pl.BlockSpec((HEADS, D), lambda i: (0, 0)),
        ],
        out_specs=[
            pl.BlockSpec((blk, D), lambda i: (i, 0)),
            pl.BlockSpec((blk, HEADS), lambda i: (i, 0)),
        ],
        out_shape=[
            jax.ShapeDtypeStruct((EH, D), jnp.float32),
            jax.ShapeDtypeStruct((EH, HEADS), jnp.float32),
        ],
    )(g, ea, wke, wve, qk, ck, wv1, bv1, rep)


def _scatter_start(i, s, c, dst_hbm, mn_hbm, ex_hbm, idx, rows, exb, semm):
    base = s * EPS + i * C2
    pltpu.sync_copy(dst_hbm.at[pl.ds(base, C2)], idx)

    @pl.when(c == 0)
    def _():
        pltpu.async_copy(mn_hbm.at[pl.ds(base, C2)], rows, semm)

    @pl.when(c == 1)
    def _():
        pltpu.async_copy(ex_hbm.at[pl.ds(base * HEADS, C2 * HEADS)],
                         exb.at[pl.ds(0, C2 * HEADS)], semm)


def _scatter_finish(i, s, c, mn_hbm, ex_hbm, acc, idx, rows, exb, semm):
    base = s * EPS + i * C2

    @pl.when(c == 0)
    def _():
        pltpu.make_async_copy(mn_hbm.at[pl.ds(base, C2)], rows, semm).wait()

    @pl.when(c == 1)
    def _():
        pltpu.make_async_copy(
            ex_hbm.at[pl.ds(base * HEADS, C2 * HEADS)],
            exb.at[pl.ds(0, C2 * HEADS)], semm).wait()
        low = lax.iota(jnp.int32, 16) < HEADS

        def expand(r, _):
            vec = jnp.where(low, exb[pl.ds(r * HEADS, 16)], 0.0)
            rows[r, pl.ds(0, 16)] = vec
            return 0

        lax.fori_loop(0, C2, expand, 0)

    pltpu.sync_copy(rows, acc.at[idx], add=True)


def _sc_scatter_body(mn_hbm, ex_hbm, dst_hbm, z_hbm, out_hbm,
                     idx0, rows0, exb0, idx1, rows1, exb1,
                     semm0, semm1, acc):
    c = lax.axis_index("c")
    s = lax.axis_index("s")
    nch = EPS // C2

    pltpu.sync_copy(z_hbm, rows0)
    pltpu.sync_copy(z_hbm, rows1)

    def zinit(j, _):
        @pl.when(j % NS == s)
        def _():
            pltpu.sync_copy(rows0, acc.at[pl.ds(j * ZC, ZC)])
        return 0

    lax.fori_loop(0, NZCH, zinit, 0)
    plsc.subcore_barrier()

    def body(i, _):
        @pl.when(i % 2 == 0)
        def _():
            _scatter_start(i, s, c, dst_hbm, mn_hbm, ex_hbm,
                           idx0, rows0, exb0, semm0)

            @pl.when(i >= 1)
            def _():
                _scatter_finish(i - 1, s, c, mn_hbm, ex_hbm, acc,
                                idx1, rows1, exb1, semm1)

        @pl.when(i % 2 == 1)
        def _():
            _scatter_start(i, s, c, dst_hbm, mn_hbm, ex_hbm,
                           idx1, rows1, exb1, semm1)

            @pl.when(i >= 1)
            def _():
                _scatter_finish(i - 1, s, c, mn_hbm, ex_hbm, acc,
                                idx0, rows0, exb0, semm0)

        return 0

    lax.fori_loop(0, nch, body, 0)
    if (nch - 1) % 2 == 0:
        _scatter_finish(nch - 1, s, c, mn_hbm, ex_hbm, acc,
                        idx0, rows0, exb0, semm0)
    else:
        _scatter_finish(nch - 1, s, c, mn_hbm, ex_hbm, acc,
                        idx1, rows1, exb1, semm1)
    plsc.subcore_barrier()

    def dump(j, _):
        @pl.when(j % NS == s)
        def _():
            r0 = j * ZC
            pltpu.sync_copy(acc.at[pl.ds(r0, ZC)], rows0)
            pltpu.sync_copy(rows0, out_hbm.at[pl.ds(c * N_DST + r0, ZC)])
        return 0

    lax.fori_loop(0, NZCH, dump, 0)


def _sc_scatter(mn, ex_flat, dst, zeros):
    mesh = plsc.VectorSubcoreMesh(core_axis_name="c", subcore_axis_name="s")
    f = functools.partial(
        pl.kernel,
        mesh=mesh,
        out_type=jax.ShapeDtypeStruct((NC * N_DST, D), jnp.float32),
        scratch_types=[
            pltpu.VMEM((C2,), jnp.int32),
            pltpu.VMEM((C2, D), jnp.float32),
            pltpu.VMEM((C2 * HEADS + 16,), jnp.float32),
            pltpu.VMEM((C2,), jnp.int32),
            pltpu.VMEM((C2, D), jnp.float32),
            pltpu.VMEM((C2 * HEADS + 16,), jnp.float32),
            pltpu.SemaphoreType.DMA,
            pltpu.SemaphoreType.DMA,
            pltpu.VMEM_SHARED((N_DST, D), jnp.float32),
        ],
    )(_sc_scatter_body)
    return f(mn, ex_flat, dst, zeros)


def _update_body(p1n_ref, p1d_ref, p2n_ref, p2d_ref, w0_ref, b0_ref,
                 w1_ref, b1_ref, rep_ref, out_ref):
    num = p1n_ref[...] + p2n_ref[...]
    den = p1d_ref[:, :HEADS] + p2d_ref[:, :HEADS]
    denw = jnp.dot(den, rep_ref[...],
                   preferred_element_type=jnp.float32) + 1e-16
    h = jnp.maximum(num / denw, 0.0)
    y0 = jnp.maximum(
        jnp.dot(h, w0_ref[...], preferred_element_type=jnp.float32)
        + b0_ref[...] + h, 0.0)
    out_ref[...] = jnp.maximum(
        jnp.dot(y0, w1_ref[...], preferred_element_type=jnp.float32)
        + b1_ref[...] + y0, 0.0)


def _tc_update(p1, p2, w0, b0, w1, b1, rep):
    blk = 1000
    grid = (N_DST // blk,)
    nb = N_DST // blk
    return pl.pallas_call(
        _update_body,
        grid=grid,
        in_specs=[
            pl.BlockSpec((blk, D), lambda i: (i, 0)),
            pl.BlockSpec((blk, D), lambda i: (i + nb, 0)),
            pl.BlockSpec((blk, D), lambda i: (i, 0)),
            pl.BlockSpec((blk, D), lambda i: (i + nb, 0)),
            pl.BlockSpec((D, D), lambda i: (0, 0)),
            pl.BlockSpec((1, D), lambda i: (0, 0)),
            pl.BlockSpec((D, D), lambda i: (0, 0)),
            pl.BlockSpec((1, D), lambda i: (0, 0)),
            pl.BlockSpec((HEADS, D), lambda i: (0, 0)),
        ],
        out_specs=pl.BlockSpec((blk, D), lambda i: (i, 0)),
        out_shape=jax.ShapeDtypeStruct((N_DST, D), jnp.float32),
    )(p1, p1, p2, p2, w0, b0, w1, b1, rep)


def kernel(x_src, x_dst, edge_attr, edge_index, q, kW0, kb0, kW1, kb1,
           vW0, vb0, vW1, vb1, oW0, ob0, oW1, ob1):
    f32 = jnp.float32
    w_src = jnp.concatenate([kW0[:D], vW0[:D]], axis=1)
    w_dst = jnp.concatenate([kW0[D:2 * D], vW0[D:2 * D]], axis=1)
    b_dst = jnp.concatenate([kb0, vb0]).reshape(1, TW)
    wke = kW0[2 * D:]
    wve = vW0[2 * D:]
    scale = np.float32(1.0 / np.sqrt(float(D_HEAD)))
    qflat = q.reshape(D)
    sel = (jnp.arange(D)[:, None] // D_HEAD
           == jnp.arange(HEADS)[None, :]).astype(f32)
    qk = scale * ((kW1 + jnp.eye(D, dtype=f32)) @ (qflat[:, None] * sel))
    ck = (scale * jnp.sum((kb1 * qflat).reshape(HEADS, D_HEAD), axis=1)
          ).reshape(1, HEADS)
    rep = sel.T

    src = edge_index[0].astype(jnp.int32)
    dst = edge_index[1].astype(jnp.int32)

    ts, td = _tc_tables(x_src, x_dst, w_src, w_dst, b_dst)
    zeros = jnp.zeros((ZC, D), f32)
    ps = []
    for h in range(2):
        sl = slice(h * EH, (h + 1) * EH)
        g = _sc_gather(ts, td, src[sl], dst[sl])
        mn, ex8 = _tc_edges(g, edge_attr[sl], wke, wve, qk, ck, vW1,
                            vb1.reshape(1, D), rep)
        ps.append(_sc_scatter(mn, ex8.reshape(EH * HEADS), dst[sl], zeros))
    return _tc_update(ps[0], ps[1], oW0, ob0.reshape(1, D),
                      oW1, ob1.reshape(1, D), rep)
